# baseline (speedup 1.0000x reference)
import functools

import jax
import jax.numpy as jnp
from jax import lax
from jax.experimental import pallas as pl
from jax.experimental.pallas import tpu as pltpu
from jax.experimental.pallas import tpu_sc as plsc

PAD = 1


@functools.partial(jax.jit, static_argnums=(2, 3, 4))
def _sc_embed(tokens_flat, table_flat, B, D, CH):
    info = plsc.get_sparse_core_info()
    NC, NS, L = info.num_cores, info.num_subcores, info.num_lanes
    NW = NC * NS
    b_per_w = B // NW
    n_ch = b_per_w // CH
    n_vec = b_per_w // L
    assert CH == L
    slabs_per_row = 8
    mesh = plsc.VectorSubcoreMesh(core_axis_name="c", subcore_axis_name="s")

    _dnums = lax.GatherDimensionNumbers(
        offset_dims=(), collapsed_slice_dims=(0,), start_index_map=(0,))

    def _vgather(x, idx):
        return lax.gather(x, idx[:, None], _dnums, (1,),
                          mode=lax.GatherScatterMode.PROMISE_IN_BOUNDS)

    def _csum16(x):
        lanes_c = lax.iota(jnp.int32, L)
        for sh in (1, 2, 4, 8):
            rolled = _vgather(x, jnp.maximum(lanes_c - sh, 0))
            x = x + jnp.where(lanes_c >= sh, rolled, 0)
        return x

    def _last_splat(x):
        return _vgather(x, jnp.zeros((L,), jnp.int32) + (L - 1))

    @functools.partial(
        pl.kernel,
        mesh=mesh,
        out_type=jax.ShapeDtypeStruct((B * D,), jnp.float32),
        scratch_types=[
            pltpu.VMEM((b_per_w,), jnp.int32),
            pltpu.VMEM((b_per_w,), jnp.int32),
            pltpu.VMEM((L,), jnp.int32),
            pltpu.VMEM((NS, L), jnp.int32),
            pltpu.VMEM_SHARED((NS, L), jnp.int32),
            pltpu.SMEM((n_ch,), jnp.int32),
            pltpu.SMEM((n_ch,), jnp.int32),
            pltpu.VMEM((CH * D,), jnp.float32),
            pltpu.VMEM((CH * D,), jnp.float32),
            pltpu.VMEM((CH * D,), jnp.float32),
            pltpu.VMEM((CH * D,), jnp.float32),
            pltpu.VMEM((D,), jnp.float32),
            pltpu.SemaphoreType.DMA,
            pltpu.SemaphoreType.DMA,
            pltpu.SemaphoreType.DMA,
            pltpu.SemaphoreType.DMA,
            pltpu.SemaphoreType.DMA,
            pltpu.SemaphoreType.DMA,
            pltpu.SemaphoreType.DMA,
            pltpu.SemaphoreType.DMA,
        ],
    )
    def k(tok_hbm, table_hbm, out_hbm, tok_v, idx_v, my_cnt_v, cnt_all_v,
          cnt_sh, flag_v, st_v, buf0, buf1, buf2, buf3, rowbuf,
          gs0, gs1, gs2, gs3, ws0, ws1, ws2, ws3):
        c = lax.axis_index("c")
        s = lax.axis_index("s")
        row = 2 * c + s // slabs_per_row
        slab = s % slabs_per_row
        base = row * (slabs_per_row * b_per_w) + slab * b_per_w

        pltpu.sync_copy(tok_hbm.at[pl.ds(base, b_per_w)], tok_v)

        def cbody(j, acc):
            t = tok_v[pl.ds(j * L, L)]
            return acc + jnp.where(t != PAD, 1, 0).astype(jnp.int32)

        acc = lax.fori_loop(0, n_vec, cbody, jnp.zeros((L,), jnp.int32))
        cnt_splat = _last_splat(_csum16(acc))

        my_cnt_v[...] = cnt_splat
        pltpu.sync_copy(my_cnt_v, cnt_sh.at[s])
        plsc.subcore_barrier()
        pltpu.sync_copy(cnt_sh, cnt_all_v)
        lanes = lax.iota(jnp.int32, L)
        cvec = jnp.zeros((L,), jnp.int32)
        for j in range(NS):
            cvec = cvec + jnp.where(lanes == j, cnt_all_v[j], 0)
        in_row = (lanes >= (s - slab)) & (lanes < s)
        start_s = _csum16(jnp.where(in_row, cvec, 0))[L - 1]

        def pbody(j, carry):
            t = tok_v[pl.ds(j * L, L)]
            v = jnp.where(t != PAD, 1, 0).astype(jnp.int32)
            cs = _csum16(v)
            idx_v[pl.ds(j * L, L)] = (cs + carry) * v + PAD
            c_last = cs[L - 1]
            flag_v[j] = (c_last == L).astype(jnp.int32)
            st_v[j] = carry + 1 + PAD
            return carry + c_last

        lax.fori_loop(0, n_vec, pbody, start_s)

        bufs = (buf0, buf1, buf2, buf3)
        gss = (gs0, gs1, gs2, gs3)
        wss = (ws0, ws1, ws2, ws3)
        NB = 4

        def is_clean(g):
            return flag_v[jnp.maximum(g, 0)] == 1

        def fire_g(g, b):
            pltpu.async_copy(table_hbm.at[pl.ds(st_v[g] * D, CH * D)],
                             bufs[b], gss[b])

        def wait_g(g, b):
            pltpu.make_async_copy(
                table_hbm.at[pl.ds(st_v[g] * D, CH * D)], bufs[b], gss[b]
            ).wait()

        def fire_w(g, b):
            pltpu.async_copy(bufs[b],
                             out_hbm.at[pl.ds((base + g * CH) * D, CH * D)],
                             wss[b])

        def wait_w(g, b):
            pltpu.make_async_copy(
                bufs[b], out_hbm.at[pl.ds((base + g * CH) * D, CH * D)],
                wss[b]).wait()

        @pl.when(is_clean(0))
        def _():
            fire_g(0, 0)

        @pl.when(is_clean(1))
        def _():
            fire_g(1, 1)

        def body(i, carry):
            for b in range(NB):
                g = i * NB + b
                dst0 = (base + g * CH) * D

                @pl.when(is_clean(g))
                def _():
                    wait_g(g, b)
                    fire_w(g, b)

                @pl.when(jnp.logical_not(is_clean(g)))
                def _():
                    iv = idx_v[pl.ds(g * CH, L)]
                    for i2 in range(CH):
                        pltpu.sync_copy(table_hbm.at[pl.ds(iv[i2] * D, D)],
                                        rowbuf)
                        pltpu.sync_copy(rowbuf,
                                        out_hbm.at[pl.ds(dst0 + i2 * D, D)])

                nb = (b + 2) % NB

                @pl.when((g - 2 >= 0) & is_clean(g - 2))
                def _():
                    wait_w(g - 2, nb)

                @pl.when((g + 2 < n_ch) & is_clean(jnp.minimum(g + 2, n_ch - 1)))
                def _():
                    fire_g(g + 2, nb)

            return carry

        lax.fori_loop(0, n_ch // NB, body, 0)

        for gt in (n_ch - 2, n_ch - 1):

            @pl.when(is_clean(gt))
            def _():
                wait_w(gt, gt % NB)

    return k(tokens_flat, table_flat)


def kernel(input, weights):
    bsz, seq_len = input.shape
    tokens = input.astype(jnp.int32)
    D = weights.shape[1]
    B = bsz * seq_len
    out = _sc_embed(tokens.reshape(B), weights.reshape(-1), B, D, 16)
    return out.reshape(bsz, seq_len, D)


# speedup vs baseline: 2.6608x; 2.6608x over previous
import functools

import jax
import jax.numpy as jnp
from jax import lax
from jax.experimental import pallas as pl
from jax.experimental.pallas import tpu as pltpu
from jax.experimental.pallas import tpu_sc as plsc

PAD = 1


@functools.partial(jax.jit, static_argnums=(2, 3, 4, 5))
def _sc_embed(tokens_flat, table, B, D, CH, SEQ):
    info = plsc.get_sparse_core_info()
    NC, NS, L = info.num_cores, info.num_subcores, info.num_lanes
    assert CH == L
    seg = SEQ // NS
    nv = seg // L
    n_sc = seg // CH
    assert n_sc % 2 == 0
    mesh = plsc.VectorSubcoreMesh(core_axis_name="c", subcore_axis_name="s")

    _dnums = lax.GatherDimensionNumbers(
        offset_dims=(), collapsed_slice_dims=(0,), start_index_map=(0,))

    def _vgather(x, idx):
        return lax.gather(x, idx[:, None], _dnums, (1,),
                          mode=lax.GatherScatterMode.PROMISE_IN_BOUNDS)

    def _csum16(x):
        lanes_c = lax.iota(jnp.int32, L)
        for sh in (1, 2, 4, 8):
            rolled = _vgather(x, jnp.maximum(lanes_c - sh, 0))
            x = x + jnp.where(lanes_c >= sh, rolled, 0)
        return x

    def _last_splat(x):
        return _vgather(x, jnp.zeros((L,), jnp.int32) + (L - 1))

    @functools.partial(
        pl.kernel,
        mesh=mesh,
        out_type=jax.ShapeDtypeStruct((B, D), jnp.float32),
        scratch_types=[
            pltpu.VMEM((2 * seg,), jnp.int32),
            pltpu.VMEM((2 * seg,), jnp.int32),
            pltpu.VMEM((L,), jnp.int32),
            pltpu.VMEM((2 * NS, L), jnp.int32),
            pltpu.VMEM_SHARED((2 * NS, L), jnp.int32),
            pltpu.SMEM((2 * nv,), jnp.int32),
            pltpu.SMEM((2 * nv,), jnp.int32),
            pltpu.VMEM((CH, D), jnp.float32),
            pltpu.VMEM((CH, D), jnp.float32),
            pltpu.VMEM((CH, D), jnp.float32),
            pltpu.VMEM((CH, D), jnp.float32),
            pltpu.SemaphoreType.DMA,
            pltpu.SemaphoreType.DMA,
            pltpu.SemaphoreType.DMA,
            pltpu.SemaphoreType.DMA,
            pltpu.SemaphoreType.DMA,
            pltpu.SemaphoreType.DMA,
            pltpu.SemaphoreType.DMA,
            pltpu.SemaphoreType.DMA,
        ],
    )
    def k(tok_hbm, table_hbm, out_hbm, tok_v, idx_v, my_cnt_v, cnt_all_v,
          cnt_sh, flags, sts, bA0, bA1, bB0, bB1,
          gA0, gA1, gB0, gB1, wA0, wA1, wB0, wB1):
        c = lax.axis_index("c")
        s = lax.axis_index("s")
        col0 = s * seg
        tb0 = (2 * c) * SEQ + col0
        tb1 = (2 * c + 1) * SEQ + col0

        pltpu.sync_copy(tok_hbm.at[pl.ds(tb0, seg)], tok_v.at[pl.ds(0, seg)])
        pltpu.sync_copy(tok_hbm.at[pl.ds(tb1, seg)], tok_v.at[pl.ds(seg, seg)])

        def cbody(j, acc):
            t = tok_v[pl.ds(j * L, L)]
            return acc + jnp.where(t != PAD, 1, 0).astype(jnp.int32)

        acc0 = lax.fori_loop(0, nv, cbody, jnp.zeros((L,), jnp.int32))
        acc1 = lax.fori_loop(nv, 2 * nv, cbody, jnp.zeros((L,), jnp.int32))

        my_cnt_v[...] = _last_splat(_csum16(acc0))
        pltpu.sync_copy(my_cnt_v, cnt_sh.at[s])
        my_cnt_v[...] = _last_splat(_csum16(acc1))
        pltpu.sync_copy(my_cnt_v, cnt_sh.at[NS + s])
        plsc.subcore_barrier()
        pltpu.sync_copy(cnt_sh, cnt_all_v)
        lanes = lax.iota(jnp.int32, L)
        cvec0 = jnp.zeros((L,), jnp.int32)
        cvec1 = jnp.zeros((L,), jnp.int32)
        for j in range(NS):
            cvec0 = cvec0 + jnp.where(lanes == j, cnt_all_v[j], 0)
            cvec1 = cvec1 + jnp.where(lanes == j, cnt_all_v[NS + j], 0)
        seg_mask = lanes < s
        start0 = _csum16(jnp.where(seg_mask, cvec0, 0))[L - 1]
        start1 = _csum16(jnp.where(seg_mask, cvec1, 0))[L - 1]

        def pbody(j, carry):
            t = tok_v[pl.ds(j * L, L)]
            v = jnp.where(t != PAD, 1, 0).astype(jnp.int32)
            cs = _csum16(v)
            idx_v[pl.ds(j * L, L)] = (cs + carry) * v + PAD
            c_last = cs[L - 1]
            flags[j] = (c_last == L).astype(jnp.int32)
            sts[j] = carry + 1 + PAD
            return carry + c_last

        lax.fori_loop(0, nv, pbody, start0)
        lax.fori_loop(nv, 2 * nv, pbody, start1)

        bufs = ((bA0, bA1), (bB0, bB1))
        gss = ((gA0, gA1), (gB0, gB1))
        wss = ((wA0, wA1), (wB0, wB1))

        def gsrc(j):
            return table_hbm.at[idx_v.at[pl.ds(j * L, L)]]

        def odst(rbase, g):
            return out_hbm.at[pl.ds(rbase + g * CH, CH)]

        def is_shared(g):
            return ((flags[g] == 1) & (flags[nv + g] == 1)
                    & (sts[g] == sts[nv + g]))

        def drain(g, p):
            sh = is_shared(g)

            @pl.when(sh)
            def _():
                pltpu.make_async_copy(
                    bufs[p][0], odst(tb1, g), wss[p][1]).wait()

            @pl.when(jnp.logical_not(sh))
            def _():
                pltpu.make_async_copy(
                    bufs[p][0], odst(tb0, g), wss[p][0]).wait()
                pltpu.make_async_copy(
                    bufs[p][1], odst(tb1, g), wss[p][1]).wait()

        def body(i, carry):
            for p in range(2):
                g = i * 2 + p
                sh = is_shared(g)

                @pl.when(g >= 2)
                def _():
                    drain(g - 2, p)

                pltpu.async_copy(gsrc(g), bufs[p][0], gss[p][0])

                @pl.when(jnp.logical_not(sh))
                def _():
                    pltpu.async_copy(gsrc(nv + g), bufs[p][1], gss[p][1])

                pltpu.make_async_copy(gsrc(g), bufs[p][0], gss[p][0]).wait()

                @pl.when(jnp.logical_not(sh))
                def _():
                    pltpu.make_async_copy(
                        gsrc(nv + g), bufs[p][1], gss[p][1]).wait()

                @pl.when(sh)
                def _():
                    pltpu.async_copy(bufs[p][0], odst(tb0, g), wss[p][0])
                    pltpu.make_async_copy(
                        bufs[p][0], odst(tb0, g), wss[p][0]).wait()
                    pltpu.async_copy(bufs[p][0], odst(tb1, g), wss[p][1])

                @pl.when(jnp.logical_not(sh))
                def _():
                    pltpu.async_copy(bufs[p][0], odst(tb0, g), wss[p][0])
                    pltpu.async_copy(bufs[p][1], odst(tb1, g), wss[p][1])

            return carry

        lax.fori_loop(0, n_sc // 2, body, 0)
        drain(n_sc - 2, 0)
        drain(n_sc - 1, 1)

    return k(tokens_flat, table)


def kernel(input, weights):
    bsz, seq_len = input.shape
    tokens = input.astype(jnp.int32)
    D = weights.shape[1]
    B = bsz * seq_len
    out = _sc_embed(tokens.reshape(B), weights, B, D, 16, seq_len)
    return out.reshape(bsz, seq_len, D)


# speedup vs baseline: 2.6789x; 1.0068x over previous
import functools

import jax
import jax.numpy as jnp
from jax import lax
from jax.experimental import pallas as pl
from jax.experimental.pallas import tpu as pltpu
from jax.experimental.pallas import tpu_sc as plsc

PAD = 1


@functools.partial(jax.jit, static_argnums=(2, 3, 4, 5))
def _sc_embed(tokens_flat, table, B, D, CH, SEQ):
    info = plsc.get_sparse_core_info()
    NC, NS, L = info.num_cores, info.num_subcores, info.num_lanes
    assert CH == L
    seg = SEQ // NS
    nv = seg // L
    n_sc = seg // CH
    assert n_sc % 2 == 0
    mesh = plsc.VectorSubcoreMesh(core_axis_name="c", subcore_axis_name="s")

    _dnums = lax.GatherDimensionNumbers(
        offset_dims=(), collapsed_slice_dims=(0,), start_index_map=(0,))

    def _vgather(x, idx):
        return lax.gather(x, idx[:, None], _dnums, (1,),
                          mode=lax.GatherScatterMode.PROMISE_IN_BOUNDS)

    def _csum16(x):
        lanes_c = lax.iota(jnp.int32, L)
        for sh in (1, 2, 4, 8):
            rolled = _vgather(x, jnp.maximum(lanes_c - sh, 0))
            x = x + jnp.where(lanes_c >= sh, rolled, 0)
        return x

    def _last_splat(x):
        return _vgather(x, jnp.zeros((L,), jnp.int32) + (L - 1))

    @functools.partial(
        pl.kernel,
        mesh=mesh,
        out_type=jax.ShapeDtypeStruct((B, D), jnp.float32),
        scratch_types=[
            pltpu.VMEM((2 * seg,), jnp.int32),
            pltpu.VMEM((2 * seg,), jnp.int32),
            pltpu.VMEM((L,), jnp.int32),
            pltpu.VMEM((2 * NS, L), jnp.int32),
            pltpu.VMEM_SHARED((2 * NS, L), jnp.int32),
            pltpu.SMEM((2 * nv,), jnp.int32),
            pltpu.SMEM((2 * nv,), jnp.int32),
            pltpu.VMEM((CH, D), jnp.float32),
            pltpu.VMEM((CH, D), jnp.float32),
            pltpu.VMEM((CH, D), jnp.float32),
            pltpu.VMEM((CH, D), jnp.float32),
            pltpu.SemaphoreType.DMA,
            pltpu.SemaphoreType.DMA,
            pltpu.SemaphoreType.DMA,
            pltpu.SemaphoreType.DMA,
            pltpu.SemaphoreType.DMA,
            pltpu.SemaphoreType.DMA,
            pltpu.SemaphoreType.DMA,
            pltpu.SemaphoreType.DMA,
        ],
    )
    def k(tok_hbm, table_hbm, out_hbm, tok_v, idx_v, my_cnt_v, cnt_all_v,
          cnt_sh, flags, sts, bA0, bA1, bB0, bB1,
          gA0, gA1, gB0, gB1, wA0, wA1, wB0, wB1):
        c = lax.axis_index("c")
        s = lax.axis_index("s")
        col0 = s * seg
        tb0 = (2 * c) * SEQ + col0
        tb1 = (2 * c + 1) * SEQ + col0

        pltpu.sync_copy(tok_hbm.at[pl.ds(tb0, seg)], tok_v.at[pl.ds(0, seg)])
        pltpu.sync_copy(tok_hbm.at[pl.ds(tb1, seg)], tok_v.at[pl.ds(seg, seg)])

        def cbody(j, acc):
            t = tok_v[pl.ds(j * L, L)]
            return acc + jnp.where(t != PAD, 1, 0).astype(jnp.int32)

        acc0 = lax.fori_loop(0, nv, cbody, jnp.zeros((L,), jnp.int32))
        acc1 = lax.fori_loop(nv, 2 * nv, cbody, jnp.zeros((L,), jnp.int32))

        my_cnt_v[...] = _last_splat(_csum16(acc0))
        pltpu.sync_copy(my_cnt_v, cnt_sh.at[s])
        my_cnt_v[...] = _last_splat(_csum16(acc1))
        pltpu.sync_copy(my_cnt_v, cnt_sh.at[NS + s])
        plsc.subcore_barrier()
        pltpu.sync_copy(cnt_sh, cnt_all_v)
        lanes = lax.iota(jnp.int32, L)
        cvec0 = jnp.zeros((L,), jnp.int32)
        cvec1 = jnp.zeros((L,), jnp.int32)
        for j in range(NS):
            cvec0 = cvec0 + jnp.where(lanes == j, cnt_all_v[j], 0)
            cvec1 = cvec1 + jnp.where(lanes == j, cnt_all_v[NS + j], 0)
        seg_mask = lanes < s
        start0 = _csum16(jnp.where(seg_mask, cvec0, 0))[L - 1]
        start1 = _csum16(jnp.where(seg_mask, cvec1, 0))[L - 1]

        def pbody(j, carry):
            t = tok_v[pl.ds(j * L, L)]
            v = jnp.where(t != PAD, 1, 0).astype(jnp.int32)
            cs = _csum16(v)
            idx_v[pl.ds(j * L, L)] = (cs + carry) * v + PAD
            c_last = cs[L - 1]
            flags[j] = (c_last == L).astype(jnp.int32)
            sts[j] = carry + 1 + PAD
            return carry + c_last

        lax.fori_loop(0, nv, pbody, start0)
        lax.fori_loop(nv, 2 * nv, pbody, start1)

        bufs = ((bA0, bA1), (bB0, bB1))
        gss = ((gA0, gA1), (gB0, gB1))
        wss = ((wA0, wA1), (wB0, wB1))

        def gsrc(j):
            return table_hbm.at[idx_v.at[pl.ds(j * L, L)]]

        def odst(rbase, g):
            return out_hbm.at[pl.ds(rbase + g * CH, CH)]

        def is_shared(g):
            return ((flags[g] == 1) & (flags[nv + g] == 1)
                    & (sts[g] == sts[nv + g]))

        def deferred_row1(g, p):
            @pl.when(is_shared(g))
            def _():
                pltpu.make_async_copy(
                    bufs[p][0], odst(tb0, g), wss[p][0]).wait()
                pltpu.async_copy(bufs[p][0], odst(tb1, g), wss[p][1])

        def drain(g, p):
            sh = is_shared(g)

            @pl.when(sh)
            def _():
                pltpu.make_async_copy(
                    bufs[p][0], odst(tb1, g), wss[p][1]).wait()

            @pl.when(jnp.logical_not(sh))
            def _():
                pltpu.make_async_copy(
                    bufs[p][0], odst(tb0, g), wss[p][0]).wait()
                pltpu.make_async_copy(
                    bufs[p][1], odst(tb1, g), wss[p][1]).wait()

        def body(i, carry):
            for p in range(2):
                g = i * 2 + p
                sh = is_shared(g)

                @pl.when(g >= 2)
                def _():
                    drain(g - 2, p)

                @pl.when(g >= 1)
                def _():
                    deferred_row1(g - 1, 1 - p)

                pltpu.async_copy(gsrc(g), bufs[p][0], gss[p][0])

                @pl.when(jnp.logical_not(sh))
                def _():
                    pltpu.async_copy(gsrc(nv + g), bufs[p][1], gss[p][1])

                pltpu.make_async_copy(gsrc(g), bufs[p][0], gss[p][0]).wait()

                @pl.when(jnp.logical_not(sh))
                def _():
                    pltpu.make_async_copy(
                        gsrc(nv + g), bufs[p][1], gss[p][1]).wait()

                pltpu.async_copy(bufs[p][0], odst(tb0, g), wss[p][0])

                @pl.when(jnp.logical_not(sh))
                def _():
                    pltpu.async_copy(bufs[p][1], odst(tb1, g), wss[p][1])

            return carry

        lax.fori_loop(0, n_sc // 2, body, 0)
        deferred_row1(n_sc - 1, 1)
        drain(n_sc - 2, 0)
        drain(n_sc - 1, 1)

    return k(tokens_flat, table)


def kernel(input, weights):
    bsz, seq_len = input.shape
    tokens = input.astype(jnp.int32)
    D = weights.shape[1]
    B = bsz * seq_len
    out = _sc_embed(tokens.reshape(B), weights, B, D, 16, seq_len)
    return out.reshape(bsz, seq_len, D)
